# baseline (speedup 1.0000x reference)
import functools

import jax
import jax.numpy as jnp
from jax import lax
from jax.experimental import pallas as pl
from jax.experimental.pallas import tpu as pltpu
from jax.experimental.pallas import tpu_sc as plsc

EMBED = 64
LABELS = 4
B = 4096
L = 50

NC, NS, LANES = 2, 16, 16
NW = NC * NS
BPW = B // NW
EPC = 2
ROWS = EPC * L
CHUNKS = BPW // EPC
SEGS = EMBED // LANES

_mesh = plsc.VectorSubcoreMesh(
    core_axis_name="c", subcore_axis_name="s", num_cores=NC, num_subcores=NS
)


def _accumulate(rows_v, slot, j, out_v):
    for e in range(EPC):
        def red(l, acc):
            r = e * L + l
            return tuple(
                acc[g] + rows_v[slot, r, pl.ds(g * LANES, LANES)]
                for g in range(SEGS)
            )
        init = tuple(jnp.zeros((LANES,), jnp.float32) for _ in range(SEGS))
        acc = lax.fori_loop(0, L, red, init)
        for g in range(SEGS):
            out_v[j * EPC + e, pl.ds(g * LANES, LANES)] = acc[g] * (1.0 / L)


@functools.partial(
    pl.kernel,
    out_type=jax.ShapeDtypeStruct((B, EMBED), jnp.float32),
    mesh=_mesh,
    scratch_types=[
        pltpu.VMEM((CHUNKS, ROWS), jnp.int32),
        pltpu.VMEM((2, ROWS, EMBED), jnp.float32),
        pltpu.VMEM((BPW, EMBED), jnp.float32),
        pltpu.SemaphoreType.DMA,
        pltpu.SemaphoreType.DMA,
    ],
    compiler_params=pltpu.CompilerParams(use_tc_tiling_on_sc=False),
)
def _embed_bag(text_hbm, table_hbm, out_hbm, idx_v, rows_v, out_v, sem0, sem1):
    wid = lax.axis_index("s") * NC + lax.axis_index("c")
    pltpu.sync_copy(text_hbm.at[wid], idx_v)

    pltpu.async_copy(table_hbm.at[idx_v.at[0]], rows_v.at[0], sem0)

    def pair_body(p, _):
        base = 2 * p
        pltpu.async_copy(table_hbm.at[idx_v.at[base + 1]], rows_v.at[1], sem1)
        pltpu.make_async_copy(
            table_hbm.at[idx_v.at[base]], rows_v.at[0], sem0
        ).wait()
        _accumulate(rows_v, 0, base, out_v)

        @pl.when(base + 2 < CHUNKS)
        def _():
            pltpu.async_copy(
                table_hbm.at[idx_v.at[base + 2]], rows_v.at[0], sem0
            )

        pltpu.make_async_copy(
            table_hbm.at[idx_v.at[base + 1]], rows_v.at[1], sem1
        ).wait()
        _accumulate(rows_v, 1, base + 1, out_v)
        return 0

    lax.fori_loop(0, CHUNKS // 2, pair_body, 0)
    pltpu.sync_copy(out_v, out_hbm.at[pl.ds(wid * BPW, BPW)])


def _head_body(emb_ref, w_ref, b_ref, out_ref):
    logits = lax.dot_general(
        emb_ref[...], w_ref[...],
        (((1,), (1,)), ((), ())),
        preferred_element_type=jnp.float32,
    ) + b_ref[...]
    out_ref[...] = 1.0 / (1.0 + jnp.exp(-logits))


_head = pl.pallas_call(
    _head_body,
    out_shape=jax.ShapeDtypeStruct((B, LABELS), jnp.float32),
)


def kernel(text, table, W, b):
    emb = _embed_bag(text.reshape(NW, CHUNKS, ROWS), table)
    return _head(emb, W, b.reshape(1, LABELS))
